# baseline (speedup 1.0000x reference)
import functools

import jax
import jax.numpy as jnp
from jax import lax
from jax.experimental import pallas as pl
from jax.experimental.pallas import tpu as pltpu
from jax.experimental.pallas import tpu_sc as plsc


_DN = (((1,), (1,)), ((), ()))


def _route_body(syms_ref, wq_ref, g_ref, cnt_ref):
    T_, N_, _ = syms_ref.shape
    wq = wq_ref[...]
    bv = lax.dot_general(wq, syms_ref[0].astype(jnp.bfloat16), _DN,
                         preferred_element_type=jnp.float32)
    bi = jnp.zeros((1, N_), jnp.int32)
    for t in range(1, T_):
        rt = lax.dot_general(wq, syms_ref[t].astype(jnp.bfloat16), _DN,
                             preferred_element_type=jnp.float32)
        upd = rt > bv
        bv = jnp.where(upd, rt, bv)
        bi = jnp.where(upd, t, bi)
    pos = lax.broadcasted_iota(jnp.int32, (1, N_), 1)
    g_ref[...] = bi * N_ + pos
    tvec = lax.broadcasted_iota(jnp.int32, (T_, 1), 0)
    oh = (bi == tvec).astype(jnp.int32)
    cnt_ref[...] = jnp.sum(oh, axis=1, keepdims=True)


def _route_call(syms16, wq16):
    T_, N_, D_ = syms16.shape
    return pl.pallas_call(
        _route_body,
        out_shape=[
            jax.ShapeDtypeStruct((1, N_), jnp.int32),
            jax.ShapeDtypeStruct((T_, 1), jnp.int32),
        ],
    )(syms16, wq16)


_NW = 32


def _sc_gather(table, g_idx, ch, tc_tiling=True):
    n_rows = g_idx.shape[0]
    lat = table.shape[1]
    b_per_w = n_rows // _NW
    nch = b_per_w // ch
    mesh = plsc.VectorSubcoreMesh(core_axis_name="core",
                                  subcore_axis_name="subcore")

    @functools.partial(
        pl.kernel,
        out_type=jax.ShapeDtypeStruct((n_rows, lat), jnp.float32),
        mesh=mesh,
        compiler_params=pltpu.CompilerParams(
            use_tc_tiling_on_sc=tc_tiling),
        scratch_types=[
            pltpu.VMEM((b_per_w,), jnp.int32),
            pltpu.VMEM((ch, lat), jnp.float32),
            pltpu.VMEM((ch, lat), jnp.float32),
            pltpu.SemaphoreType.DMA,
            pltpu.SemaphoreType.DMA,
            pltpu.SemaphoreType.DMA,
            pltpu.SemaphoreType.DMA,
        ],
    )
    def gk(x_hbm, i_hbm, o_hbm, idx_v, rows0, rows1, si0, si1, so0, so1):
        wid = lax.axis_index("subcore") * 2 + lax.axis_index("core")
        base = wid * b_per_w
        pltpu.sync_copy(i_hbm.at[pl.ds(base, b_per_w)], idx_v)

        bufs = (rows0, rows1)
        sis = (si0, si1)
        sos = (so0, so1)
        gin = [None] * nch
        gout = [None] * nch
        for c in range(nch):
            b = c % 2
            if c >= 2:
                gout[c - 2].wait()
            gin[c] = pltpu.async_copy(
                x_hbm.at[idx_v.at[pl.ds(c * ch, ch)]], bufs[b], sis[b])
            if c >= 1:
                gin[c - 1].wait()
                gout[c - 1] = pltpu.async_copy(
                    bufs[1 - b], o_hbm.at[pl.ds(base + (c - 1) * ch, ch)],
                    sos[1 - b])
        gin[nch - 1].wait()
        gout[nch - 1] = pltpu.async_copy(
            bufs[(nch - 1) % 2],
            o_hbm.at[pl.ds(base + (nch - 1) * ch, ch)], sos[(nch - 1) % 2])
        if nch >= 2:
            gout[nch - 2].wait()
        gout[nch - 1].wait()

    return gk(table, g_idx)


_RB = 512
_CK = 1024


def _dense_body(ts_ref, bc_ref, wr16_ref, br_ref, ws16_ref, bs_ref,
                wc1a16_ref, wc1b16_ref, bc1_ref, wc216_ref, bc2_ref,
                cb16_ref, csq_ref,
                out_ref, idx_ref):
    f32 = jnp.float32
    bf16 = jnp.bfloat16
    ts = ts_ref[...]
    z16 = jnp.concatenate([ts, bc_ref[...]], axis=1).astype(bf16)
    zr = lax.dot_general(z16, wr16_ref[...], _DN,
                         preferred_element_type=f32) + br_ref[...]
    zr16 = zr.astype(bf16)
    raw = lax.dot_general(zr16, ws16_ref[...], _DN,
                          preferred_element_type=f32) + bs_ref[...]
    R = raw.shape[0]
    ncb = cb16_ref.shape[0]
    xsq = jnp.sum(raw * raw, axis=1, keepdims=True)
    raw16 = raw.astype(bf16)
    lane = lax.broadcasted_iota(jnp.int32, (R, _CK), 1)
    dn_nt = (((1,), (0,)), ((), ()))
    best = jnp.full((R, 1), jnp.inf, f32)
    bidx = jnp.zeros((R, 1), jnp.int32)
    q = jnp.zeros((R, cb16_ref.shape[1]), f32)
    for k in range(ncb // _CK):
        cb16 = cb16_ref[k * _CK:(k + 1) * _CK, :]
        csq = csq_ref[0:1, k * _CK:(k + 1) * _CK]
        d2 = (xsq + csq) - 2.0 * lax.dot_general(
            raw16, cb16, _DN, preferred_element_type=f32)
        m = jnp.min(d2, axis=1, keepdims=True)
        a = jnp.min(jnp.where(d2 <= m, lane, _CK), axis=1, keepdims=True)
        oh = (lane == a).astype(bf16)
        qk = lax.dot_general(oh, cb16, dn_nt, preferred_element_type=f32)
        upd = m < best
        best = jnp.where(upd, m, best)
        bidx = jnp.where(upd, a + k * _CK, bidx)
        q = jnp.where(upd, qk, q)
    h = (lax.dot_general(zr16, wc1a16_ref[...], _DN,
                         preferred_element_type=f32)
         + lax.dot_general(q.astype(bf16), wc1b16_ref[...], _DN,
                           preferred_element_type=f32)
         + bc1_ref[...])
    h16 = jnp.maximum(h, 0.0).astype(bf16)
    out = (lax.dot_general(h16, wc216_ref[...], _DN,
                           preferred_element_type=f32)
           + bc2_ref[...] + ts)
    out_ref[...] = out
    idx_ref[...] = bidx


def _dense_call(ts, bc, wr16, br2, ws16, bs2, wc1a16, wc1b16, bc12, wc216,
                bc22, cb16, csq):
    n_, lat = ts.shape
    grid = (n_ // _RB,)
    row_spec = pl.BlockSpec((_RB, lat), lambda i: (i, 0))
    full = lambda arr: pl.BlockSpec(arr.shape, lambda i: (0, 0))
    return pl.pallas_call(
        _dense_body,
        grid=grid,
        in_specs=[
            row_spec,
            row_spec,
            full(wr16), full(br2), full(ws16), full(bs2),
            full(wc1a16), full(wc1b16), full(bc12), full(wc216), full(bc22),
            full(cb16), full(csq),
        ],
        out_specs=[
            row_spec,
            pl.BlockSpec((_RB, 1), lambda i: (i, 0)),
        ],
        out_shape=[
            jax.ShapeDtypeStruct((n_, lat), jnp.float32),
            jax.ShapeDtypeStruct((n_, 1), jnp.int32),
        ],
    )(ts, bc, wr16, br2, ws16, bs2, wc1a16, wc1b16, bc12, wc216, bc22,
      cb16, csq)


def kernel(token_state, bus_symbols, bus_indices, bus_outputs, bus_mask,
           Wq, bq, Wr, br, Ws, bs, Wc1, bc1, Wc2, bc2, codebook):
    Bb, Ss, lat = token_state.shape
    T_, _, _, D_ = bus_symbols.shape
    N_ = Bb * Ss

    bf16 = jnp.bfloat16
    syms = bus_symbols.reshape(T_, N_, D_)
    table = bus_outputs.reshape(T_ * N_, lat)

    g_idx, counts = _route_call(syms, Wq.astype(bf16))
    keep_mask = counts[:, 0] == 0

    bus_context = _sc_gather(table, g_idx.reshape(N_), ch=32)

    csq = jnp.sum(codebook ** 2, axis=1).reshape(1, -1)
    cb16 = codebook.astype(bf16)
    node_out, idx = _dense_call(
        token_state.reshape(N_, lat), bus_context,
        Wr.astype(bf16), br.reshape(1, lat),
        Ws.astype(bf16), bs.reshape(1, D_),
        Wc1[:, :lat].astype(bf16), Wc1[:, lat:].astype(bf16),
        bc1.reshape(1, lat),
        Wc2.astype(bf16), bc2.reshape(1, lat),
        cb16, csq)
    quant = _sc_gather(codebook, idx.reshape(N_), ch=128,
                       tc_tiling=False)

    return (node_out.reshape(Bb, Ss, lat),
            quant.reshape(Bb, Ss, D_),
            idx.reshape(Bb, Ss),
            keep_mask)


# speedup vs baseline: 1.0222x; 1.0222x over previous
import functools

import jax
import jax.numpy as jnp
from jax import lax
from jax.experimental import pallas as pl
from jax.experimental.pallas import tpu as pltpu
from jax.experimental.pallas import tpu_sc as plsc


_DN = (((1,), (1,)), ((), ()))


def _route_body(syms_ref, wq_ref, g_ref, cnt_ref):
    T_, N_, _ = syms_ref.shape
    wq = wq_ref[...]
    bv = lax.dot_general(wq, syms_ref[0], _DN,
                         preferred_element_type=jnp.float32)
    bi = jnp.zeros((1, N_), jnp.int32)
    for t in range(1, T_):
        rt = lax.dot_general(wq, syms_ref[t], _DN,
                             preferred_element_type=jnp.float32)
        upd = rt > bv
        bv = jnp.where(upd, rt, bv)
        bi = jnp.where(upd, t, bi)
    pos = lax.broadcasted_iota(jnp.int32, (1, N_), 1)
    g_ref[...] = bi * N_ + pos
    tvec = lax.broadcasted_iota(jnp.int32, (T_, 1), 0)
    oh = (bi == tvec).astype(jnp.int32)
    cnt_ref[...] = jnp.sum(oh, axis=1, keepdims=True)


def _route_call(syms16, wq16):
    T_, N_, D_ = syms16.shape
    return pl.pallas_call(
        _route_body,
        out_shape=[
            jax.ShapeDtypeStruct((1, N_), jnp.int32),
            jax.ShapeDtypeStruct((T_, 1), jnp.int32),
        ],
    )(syms16, wq16)


_NW = 32


def _sc_gather(table, g_idx, ch, tc_tiling=True):
    n_rows = g_idx.shape[0]
    lat = table.shape[1]
    b_per_w = n_rows // _NW
    nch = b_per_w // ch
    mesh = plsc.VectorSubcoreMesh(core_axis_name="core",
                                  subcore_axis_name="subcore")

    @functools.partial(
        pl.kernel,
        out_type=jax.ShapeDtypeStruct((n_rows, lat), jnp.float32),
        mesh=mesh,
        compiler_params=pltpu.CompilerParams(
            use_tc_tiling_on_sc=tc_tiling),
        scratch_types=[
            pltpu.VMEM((b_per_w,), jnp.int32),
            pltpu.VMEM((ch, lat), jnp.float32),
            pltpu.VMEM((ch, lat), jnp.float32),
            pltpu.SemaphoreType.DMA,
            pltpu.SemaphoreType.DMA,
            pltpu.SemaphoreType.DMA,
            pltpu.SemaphoreType.DMA,
        ],
    )
    def gk(x_hbm, i_hbm, o_hbm, idx_v, rows0, rows1, si0, si1, so0, so1):
        wid = lax.axis_index("subcore") * 2 + lax.axis_index("core")
        base = wid * b_per_w
        pltpu.sync_copy(i_hbm.at[pl.ds(base, b_per_w)], idx_v)

        bufs = (rows0, rows1)
        sis = (si0, si1)
        sos = (so0, so1)
        gin = [None] * nch
        gout = [None] * nch
        for c in range(nch):
            b = c % 2
            if c >= 2:
                gout[c - 2].wait()
            gin[c] = pltpu.async_copy(
                x_hbm.at[idx_v.at[pl.ds(c * ch, ch)]], bufs[b], sis[b])
            if c >= 1:
                gin[c - 1].wait()
                gout[c - 1] = pltpu.async_copy(
                    bufs[1 - b], o_hbm.at[pl.ds(base + (c - 1) * ch, ch)],
                    sos[1 - b])
        gin[nch - 1].wait()
        gout[nch - 1] = pltpu.async_copy(
            bufs[(nch - 1) % 2],
            o_hbm.at[pl.ds(base + (nch - 1) * ch, ch)], sos[(nch - 1) % 2])
        if nch >= 2:
            gout[nch - 2].wait()
        gout[nch - 1].wait()

    return gk(table, g_idx)


_RB = 512
_CK = 1024


def _dense_body(ts_ref, bc_ref, wr16_ref, br_ref, ws16_ref, bs_ref,
                wc1a16_ref, wc1b16_ref, bc1_ref, wc216_ref, bc2_ref,
                cb16_ref, csq_ref,
                out_ref, idx_ref):
    f32 = jnp.float32
    bf16 = jnp.bfloat16
    ts = ts_ref[...]
    z16 = jnp.concatenate([ts, bc_ref[...]], axis=1).astype(bf16)
    zr = lax.dot_general(z16, wr16_ref[...], _DN,
                         preferred_element_type=f32) + br_ref[...]
    zr16 = zr.astype(bf16)
    raw = lax.dot_general(zr16, ws16_ref[...], _DN,
                          preferred_element_type=f32) + bs_ref[...]
    R = raw.shape[0]
    ncb = cb16_ref.shape[0]
    xsq = jnp.sum(raw * raw, axis=1, keepdims=True)
    raw16 = raw.astype(bf16)
    lane = lax.broadcasted_iota(jnp.int32, (R, _CK), 1)
    dn_nt = (((1,), (0,)), ((), ()))
    best = jnp.full((R, 1), jnp.inf, f32)
    bidx = jnp.zeros((R, 1), jnp.int32)
    q = jnp.zeros((R, cb16_ref.shape[1]), f32)
    for k in range(ncb // _CK):
        cb16 = cb16_ref[k * _CK:(k + 1) * _CK, :]
        csq = csq_ref[0:1, k * _CK:(k + 1) * _CK]
        d2 = (xsq + csq) - 2.0 * lax.dot_general(
            raw16, cb16, _DN, preferred_element_type=f32)
        m = jnp.min(d2, axis=1, keepdims=True)
        a = jnp.min(jnp.where(d2 <= m, lane, _CK), axis=1, keepdims=True)
        oh = (lane == a).astype(bf16)
        qk = lax.dot_general(oh, cb16, dn_nt, preferred_element_type=f32)
        upd = m < best
        best = jnp.where(upd, m, best)
        bidx = jnp.where(upd, a + k * _CK, bidx)
        q = jnp.where(upd, qk, q)
    h = (lax.dot_general(zr16, wc1a16_ref[...], _DN,
                         preferred_element_type=f32)
         + lax.dot_general(q.astype(bf16), wc1b16_ref[...], _DN,
                           preferred_element_type=f32)
         + bc1_ref[...])
    h16 = jnp.maximum(h, 0.0).astype(bf16)
    out = (lax.dot_general(h16, wc216_ref[...], _DN,
                           preferred_element_type=f32)
           + bc2_ref[...] + ts)
    out_ref[...] = out
    idx_ref[...] = bidx


def _dense_call(ts, bc, wr16, br2, ws16, bs2, wc1a16, wc1b16, bc12, wc216,
                bc22, cb16, csq):
    n_, lat = ts.shape
    grid = (n_ // _RB,)
    row_spec = pl.BlockSpec((_RB, lat), lambda i: (i, 0))
    full = lambda arr: pl.BlockSpec(arr.shape, lambda i: (0, 0))
    return pl.pallas_call(
        _dense_body,
        grid=grid,
        in_specs=[
            row_spec,
            row_spec,
            full(wr16), full(br2), full(ws16), full(bs2),
            full(wc1a16), full(wc1b16), full(bc12), full(wc216), full(bc22),
            full(cb16), full(csq),
        ],
        out_specs=[
            row_spec,
            pl.BlockSpec((_RB, 1), lambda i: (i, 0)),
        ],
        out_shape=[
            jax.ShapeDtypeStruct((n_, lat), jnp.float32),
            jax.ShapeDtypeStruct((n_, 1), jnp.int32),
        ],
    )(ts, bc, wr16, br2, ws16, bs2, wc1a16, wc1b16, bc12, wc216, bc22,
      cb16, csq)


def kernel(token_state, bus_symbols, bus_indices, bus_outputs, bus_mask,
           Wq, bq, Wr, br, Ws, bs, Wc1, bc1, Wc2, bc2, codebook):
    Bb, Ss, lat = token_state.shape
    T_, _, _, D_ = bus_symbols.shape
    N_ = Bb * Ss

    bf16 = jnp.bfloat16
    syms16 = bus_symbols.reshape(T_, N_, D_).astype(bf16)
    table = bus_outputs.reshape(T_ * N_, lat)

    g_idx, counts = _route_call(syms16, Wq.astype(bf16))
    keep_mask = counts[:, 0] == 0

    bus_context = _sc_gather(table, g_idx.reshape(N_), ch=32)

    csq = jnp.sum(codebook ** 2, axis=1).reshape(1, -1)
    cb16 = codebook.astype(bf16)
    node_out, idx = _dense_call(
        token_state.reshape(N_, lat), bus_context,
        Wr.astype(bf16), br.reshape(1, lat),
        Ws.astype(bf16), bs.reshape(1, D_),
        Wc1[:, :lat].astype(bf16), Wc1[:, lat:].astype(bf16),
        bc1.reshape(1, lat),
        Wc2.astype(bf16), bc2.reshape(1, lat),
        cb16, csq)
    quant = _sc_gather(codebook, idx.reshape(N_), ch=128,
                       tc_tiling=False)

    return (node_out.reshape(Bb, Ss, lat),
            quant.reshape(Bb, Ss, D_),
            idx.reshape(Bb, Ss),
            keep_mask)
